# speedup vs baseline: 25.6743x; 25.6743x over previous
import functools

import jax
import jax.numpy as jnp
from jax import lax
from jax.experimental import pallas as pl
from jax.experimental.pallas import tpu as pltpu
from jax.experimental.pallas import tpu_sc as plsc

NNZ = 1048576
D0, D1, D2 = 64, 4096, 256
SLICE_IDX = 10
OUT_N = D1 * D2

NUM_CORES = 2
NUM_SUBCORES = 16
NUM_TILES = NUM_CORES * NUM_SUBCORES
PER_TILE = NNZ // NUM_TILES
CHUNK = 2048
ROWS = CHUNK // 128
PER_SUB = OUT_N // NUM_SUBCORES
ZBUF = 4096

_mesh = plsc.VectorSubcoreMesh(core_axis_name="c", subcore_axis_name="s")


@functools.partial(
    pl.kernel,
    out_type=jax.ShapeDtypeStruct((NUM_CORES, OUT_N), jnp.float32),
    mesh=_mesh,
    scratch_types=[
        pltpu.VMEM((CHUNK,), jnp.int32),
        pltpu.VMEM((CHUNK,), jnp.int32),
        pltpu.VMEM((CHUNK,), jnp.int32),
        pltpu.VMEM((CHUNK,), jnp.float32),
        pltpu.VMEM((ROWS, 128), jnp.int32),
        pltpu.VMEM((ROWS, 128), jnp.float32),
        pltpu.VMEM((ZBUF,), jnp.float32),
        pltpu.VMEM_SHARED((OUT_N,), jnp.float32),
        pltpu.SemaphoreType.DMA,
    ],
)
def _sc_scatter(idx0_hbm, idx1_hbm, idx2_hbm, vals_hbm, out_hbm,
                b0, b1, b2, bv, lin2d, val2d, zbuf, accum, sem):
    c = lax.axis_index("c")
    s = lax.axis_index("s")

    zero16 = jnp.zeros((16,), jnp.float32)

    @pl.loop(0, ZBUF, step=16)
    def _(i):
        zbuf[pl.ds(i, 16)] = zero16

    @pl.loop(0, PER_SUB, step=ZBUF)
    def _(k):
        pltpu.sync_copy(zbuf, accum.at[pl.ds(s * PER_SUB + k, ZBUF)])

    plsc.subcore_barrier()

    base = (c * NUM_SUBCORES + s) * PER_TILE

    @pl.loop(0, PER_TILE, step=CHUNK)
    def _(off):
        g = base + off
        cp0 = pltpu.async_copy(idx0_hbm.at[pl.ds(g, CHUNK)], b0, sem)
        cp1 = pltpu.async_copy(idx1_hbm.at[pl.ds(g, CHUNK)], b1, sem)
        cp2 = pltpu.async_copy(idx2_hbm.at[pl.ds(g, CHUNK)], b2, sem)
        cp3 = pltpu.async_copy(vals_hbm.at[pl.ds(g, CHUNK)], bv, sem)
        cp0.wait()
        cp1.wait()
        cp2.wait()
        cp3.wait()

        @pl.loop(0, ROWS)
        def _(r):
            @pl.loop(0, 128, step=16)
            def _(col):
                co = r * 128 + col
                sl = pl.ds(co, 16)
                m = b0[sl] == SLICE_IDX
                lin = b1[sl] * D2 + b2[sl]
                val = jnp.where(m, bv[sl], 0.0)
                lin2d[r, pl.ds(col, 16)] = lin
                val2d[r, pl.ds(col, 16)] = val

        for r in range(ROWS):
            pltpu.sync_copy(val2d.at[r], accum.at[lin2d.at[r]], add=True)

    plsc.subcore_barrier()

    pltpu.sync_copy(accum.at[pl.ds(s * PER_SUB, PER_SUB)],
                    out_hbm.at[c, pl.ds(s * PER_SUB, PER_SUB)])


def _combine_body(p_ref, o_ref):
    o_ref[...] = p_ref[0] + p_ref[1]


def kernel(idx0, idx1, idx2, values):
    partials = _sc_scatter(idx0, idx1, idx2, values)
    p = partials.reshape(NUM_CORES, D1, D2)
    out = pl.pallas_call(
        _combine_body,
        grid=(8,),
        in_specs=[pl.BlockSpec((NUM_CORES, D1 // 8, D2), lambda i: (0, i, 0))],
        out_specs=pl.BlockSpec((D1 // 8, D2), lambda i: (i, 0)),
        out_shape=jax.ShapeDtypeStruct((D1, D2), jnp.float32),
    )(p)
    return out.reshape(1, D1, D2)


# speedup vs baseline: 35.7457x; 1.3923x over previous
import functools

import jax
import jax.numpy as jnp
from jax import lax
from jax.experimental import pallas as pl
from jax.experimental.pallas import tpu as pltpu
from jax.experimental.pallas import tpu_sc as plsc

NNZ = 1048576
D0, D1, D2 = 64, 4096, 256
SLICE_IDX = 10
OUT_N = D1 * D2

NUM_CORES = 2
NUM_SUBCORES = 16
NUM_TILES = NUM_CORES * NUM_SUBCORES
PER_TILE = NNZ // NUM_TILES
CHUNK = 4096
ROWS = CHUNK // 128
NCHUNK = PER_TILE // CHUNK
PER_SUB = OUT_N // NUM_SUBCORES
ZBUF = 4096

_mesh = plsc.VectorSubcoreMesh(core_axis_name="c", subcore_axis_name="s")


@functools.partial(
    pl.kernel,
    out_type=jax.ShapeDtypeStruct((NUM_CORES, OUT_N), jnp.float32),
    mesh=_mesh,
    scratch_types=[
        pltpu.VMEM((2, CHUNK), jnp.int32),
        pltpu.VMEM((2, CHUNK), jnp.int32),
        pltpu.VMEM((2, CHUNK), jnp.int32),
        pltpu.VMEM((2, CHUNK), jnp.float32),
        pltpu.VMEM((CHUNK,), jnp.int32),
        pltpu.VMEM((CHUNK,), jnp.int32),
        pltpu.VMEM((CHUNK,), jnp.float32),
        pltpu.VMEM((CHUNK,), jnp.float32),
        pltpu.VMEM((ZBUF,), jnp.float32),
        pltpu.VMEM_SHARED((OUT_N,), jnp.float32),
        pltpu.SemaphoreType.DMA,
        pltpu.SemaphoreType.DMA,
        pltpu.SemaphoreType.DMA,
        pltpu.SemaphoreType.DMA,
    ],
)
def _sc_scatter(idx0_hbm, idx1_hbm, idx2_hbm, vals_hbm, out_hbm,
                b0, b1, b2, bv, lin_a, lin_b, val_a, val_b, zbuf, accum,
                sem_a, sem_b, ssem_a, ssem_b):
    c = lax.axis_index("c")
    s = lax.axis_index("s")

    zero16 = jnp.zeros((16,), jnp.float32)

    @pl.loop(0, ZBUF, step=16)
    def _(i):
        zbuf[pl.ds(i, 16)] = zero16

    @pl.loop(0, PER_SUB, step=ZBUF)
    def _(k):
        pltpu.sync_copy(zbuf, accum.at[pl.ds(s * PER_SUB + k, ZBUF)])

    plsc.subcore_barrier()

    base = (c * NUM_SUBCORES + s) * PER_TILE
    in_sems = (sem_a, sem_b)
    sc_sems = (ssem_a, ssem_b)
    lin_bufs = (lin_a, lin_b)
    val_bufs = (val_a, val_b)

    def issue_in(k, b):
        g = base + k * CHUNK
        return [
            pltpu.async_copy(idx0_hbm.at[pl.ds(g, CHUNK)], b0.at[b], in_sems[b]),
            pltpu.async_copy(idx1_hbm.at[pl.ds(g, CHUNK)], b1.at[b], in_sems[b]),
            pltpu.async_copy(idx2_hbm.at[pl.ds(g, CHUNK)], b2.at[b], in_sems[b]),
            pltpu.async_copy(vals_hbm.at[pl.ds(g, CHUNK)], bv.at[b], in_sems[b]),
        ]

    pending_in = {0: issue_in(0, 0)}
    pending_sc = {}
    for k in range(NCHUNK):
        b = k % 2
        for cp in pending_in.pop(k):
            cp.wait()
        if k + 1 < NCHUNK:
            pending_in[k + 1] = issue_in(k + 1, (k + 1) % 2)
        if k - 2 in pending_sc:
            pending_sc.pop(k - 2).wait()

        lin_buf = lin_bufs[b]
        val_buf = val_bufs[b]

        @pl.loop(0, CHUNK, step=16)
        def _(co):
            sl = pl.ds(co, 16)
            m = b0[b, sl] == SLICE_IDX
            lin = b1[b, sl] * D2 + b2[b, sl]
            val = jnp.where(m, bv[b, sl], 0.0)
            lin_buf[sl] = lin
            val_buf[sl] = val

        pending_sc[k] = pltpu.async_copy(
            val_buf, accum.at[lin_buf], sc_sems[b], add=True)

    for h in pending_sc.values():
        h.wait()

    plsc.subcore_barrier()

    pltpu.sync_copy(accum.at[pl.ds(s * PER_SUB, PER_SUB)],
                    out_hbm.at[c, pl.ds(s * PER_SUB, PER_SUB)])


def _combine_body(p_ref, o_ref):
    o_ref[...] = p_ref[0] + p_ref[1]


def kernel(idx0, idx1, idx2, values):
    partials = _sc_scatter(idx0, idx1, idx2, values)
    p = partials.reshape(NUM_CORES, D1, D2)
    out = pl.pallas_call(
        _combine_body,
        grid=(8,),
        in_specs=[pl.BlockSpec((NUM_CORES, D1 // 8, D2), lambda i: (0, i, 0))],
        out_specs=pl.BlockSpec((D1 // 8, D2), lambda i: (i, 0)),
        out_shape=jax.ShapeDtypeStruct((D1, D2), jnp.float32),
    )(p)
    return out.reshape(1, D1, D2)


# speedup vs baseline: 37.4877x; 1.0487x over previous
import functools

import jax
import jax.numpy as jnp
from jax import lax
from jax.experimental import pallas as pl
from jax.experimental.pallas import tpu as pltpu
from jax.experimental.pallas import tpu_sc as plsc

NNZ = 1048576
D0, D1, D2 = 64, 4096, 256
SLICE_IDX = 10
OUT_N = D1 * D2

NUM_CORES = 2
NUM_SUBCORES = 16
NUM_TILES = NUM_CORES * NUM_SUBCORES
PER_TILE = NNZ // NUM_TILES
CHUNK = 4096
ROWS = CHUNK // 128
NCHUNK = PER_TILE // CHUNK
PER_SUB = OUT_N // NUM_SUBCORES
ZBUF = 4096

_mesh = plsc.VectorSubcoreMesh(core_axis_name="c", subcore_axis_name="s")


@functools.partial(
    pl.kernel,
    out_type=jax.ShapeDtypeStruct((2 * OUT_N,), jnp.float32),
    mesh=_mesh,
    scratch_types=[
        pltpu.VMEM((2, CHUNK), jnp.int32),
        pltpu.VMEM((2, CHUNK), jnp.int32),
        pltpu.VMEM((2, CHUNK), jnp.int32),
        pltpu.VMEM((2, CHUNK), jnp.float32),
        pltpu.VMEM((CHUNK,), jnp.int32),
        pltpu.VMEM((CHUNK,), jnp.int32),
        pltpu.VMEM((CHUNK,), jnp.float32),
        pltpu.VMEM((CHUNK,), jnp.float32),
        pltpu.VMEM((ZBUF,), jnp.float32),
        pltpu.VMEM_SHARED((OUT_N,), jnp.float32),
        pltpu.SemaphoreType.DMA,
        pltpu.SemaphoreType.DMA,
        pltpu.SemaphoreType.DMA,
        pltpu.SemaphoreType.DMA,
    ],
)
def _sc_scatter(idx0_hbm, idx1_hbm, idx2_hbm, vals_hbm, out_hbm,
                b0, b1, b2, bv, lin_a, lin_b, val_a, val_b, zbuf, accum,
                sem_a, sem_b, ssem_a, ssem_b):
    c = lax.axis_index("c")
    s = lax.axis_index("s")

    zero16 = jnp.zeros((16,), jnp.float32)

    @pl.loop(0, ZBUF, step=16)
    def _(i):
        zbuf[pl.ds(i, 16)] = zero16

    @pl.loop(0, PER_SUB, step=ZBUF)
    def _(k):
        pltpu.sync_copy(zbuf, accum.at[pl.ds(s * PER_SUB + k, ZBUF)])

    plsc.subcore_barrier()

    base = (c * NUM_SUBCORES + s) * PER_TILE
    in_sems = (sem_a, sem_b)
    sc_sems = (ssem_a, ssem_b)
    lin_bufs = (lin_a, lin_b)
    val_bufs = (val_a, val_b)

    def issue_in(k, b):
        g = base + k * CHUNK
        return [
            pltpu.async_copy(idx0_hbm.at[pl.ds(g, CHUNK)], b0.at[b], in_sems[b]),
            pltpu.async_copy(idx1_hbm.at[pl.ds(g, CHUNK)], b1.at[b], in_sems[b]),
            pltpu.async_copy(idx2_hbm.at[pl.ds(g, CHUNK)], b2.at[b], in_sems[b]),
            pltpu.async_copy(vals_hbm.at[pl.ds(g, CHUNK)], bv.at[b], in_sems[b]),
        ]

    pending_in = {0: issue_in(0, 0)}
    pending_sc = {}
    for k in range(NCHUNK):
        b = k % 2
        for cp in pending_in.pop(k):
            cp.wait()
        if k + 1 < NCHUNK:
            pending_in[k + 1] = issue_in(k + 1, (k + 1) % 2)
        if k - 2 in pending_sc:
            pending_sc.pop(k - 2).wait()

        lin_buf = lin_bufs[b]
        val_buf = val_bufs[b]

        @pl.loop(0, CHUNK, step=16)
        def _(co):
            sl = pl.ds(co, 16)
            m = b0[b, sl] == SLICE_IDX
            lin = b1[b, sl] * D2 + b2[b, sl]
            val = jnp.where(m, bv[b, sl], 0.0)
            lin_buf[sl] = lin
            val_buf[sl] = val

        pending_sc[k] = pltpu.async_copy(
            val_buf, accum.at[lin_buf], sc_sems[b], add=True)

    for h in pending_sc.values():
        h.wait()

    plsc.subcore_barrier()

    pltpu.sync_copy(accum.at[pl.ds(s * PER_SUB, PER_SUB)],
                    out_hbm.at[pl.ds(c * OUT_N + s * PER_SUB, PER_SUB)])


def _combine_body(a_ref, b_ref, o_ref):
    s = a_ref[...] + b_ref[...]
    o_ref[...] = s.reshape(o_ref.shape)


def kernel(idx0, idx1, idx2, values):
    partials = _sc_scatter(idx0, idx1, idx2, values)
    p0 = partials[:OUT_N]
    p1 = partials[OUT_N:]
    a = p0.reshape(D1 * D2 // 128, 128)
    b = p1.reshape(D1 * D2 // 128, 128)
    nblk = 8
    out = pl.pallas_call(
        _combine_body,
        grid=(nblk,),
        in_specs=[
            pl.BlockSpec((D1 * D2 // 128 // nblk, 128), lambda i: (i, 0)),
            pl.BlockSpec((D1 * D2 // 128 // nblk, 128), lambda i: (i, 0)),
        ],
        out_specs=pl.BlockSpec((D1 // nblk, D2), lambda i: (i, 0)),
        out_shape=jax.ShapeDtypeStruct((D1, D2), jnp.float32),
    )(a, b)
    return out.reshape(1, D1, D2)


# speedup vs baseline: 40.2581x; 1.0739x over previous
import functools

import jax
import jax.numpy as jnp
from jax import lax
from jax.experimental import pallas as pl
from jax.experimental.pallas import tpu as pltpu
from jax.experimental.pallas import tpu_sc as plsc

NNZ = 1048576
D0, D1, D2 = 64, 4096, 256
SLICE_IDX = 10
OUT_N = D1 * D2

NUM_CORES = 2
NUM_SUBCORES = 16
NUM_TILES = NUM_CORES * NUM_SUBCORES
PER_TILE = NNZ // NUM_TILES
CHUNK = 4096
ROWS = CHUNK // 128
NCHUNK = PER_TILE // CHUNK
PER_SUB = OUT_N // NUM_SUBCORES
ZBUF = 4096

_mesh = plsc.VectorSubcoreMesh(core_axis_name="c", subcore_axis_name="s")


@functools.partial(
    pl.kernel,
    out_type=jax.ShapeDtypeStruct((2 * OUT_N,), jnp.float32),
    mesh=_mesh,
    scratch_types=[
        pltpu.VMEM((2, CHUNK), jnp.int32),
        pltpu.VMEM((2, CHUNK), jnp.int32),
        pltpu.VMEM((2, CHUNK), jnp.int32),
        pltpu.VMEM((2, CHUNK), jnp.float32),
        pltpu.VMEM((CHUNK,), jnp.int32),
        pltpu.VMEM((CHUNK,), jnp.int32),
        pltpu.VMEM((CHUNK,), jnp.float32),
        pltpu.VMEM((CHUNK,), jnp.float32),
        pltpu.VMEM((ZBUF,), jnp.float32),
        pltpu.VMEM_SHARED((OUT_N,), jnp.float32),
        pltpu.SemaphoreType.DMA,
        pltpu.SemaphoreType.DMA,
        pltpu.SemaphoreType.DMA,
        pltpu.SemaphoreType.DMA,
    ],
)
def _sc_scatter(idx0_hbm, idx1_hbm, idx2_hbm, vals_hbm, out_hbm,
                b0, b1, b2, bv, lin_a, lin_b, val_a, val_b, zbuf, accum,
                sem_a, sem_b, ssem_a, ssem_b):
    c = lax.axis_index("c")
    s = lax.axis_index("s")

    zero16 = jnp.zeros((16,), jnp.float32)

    @pl.loop(0, ZBUF, step=16)
    def _(i):
        zbuf[pl.ds(i, 16)] = zero16

    @pl.loop(0, PER_SUB, step=ZBUF)
    def _(k):
        pltpu.sync_copy(zbuf, accum.at[pl.ds(s * PER_SUB + k, ZBUF)])

    plsc.subcore_barrier()

    base = (c * NUM_SUBCORES + s) * PER_TILE
    in_sems = (sem_a, sem_b)
    sc_sems = (ssem_a, ssem_b)
    lin_bufs = (lin_a, lin_b)
    val_bufs = (val_a, val_b)

    def issue_in(k, b):
        g = base + k * CHUNK
        return [
            pltpu.async_copy(idx0_hbm.at[pl.ds(g, CHUNK)], b0.at[b], in_sems[b]),
            pltpu.async_copy(idx1_hbm.at[pl.ds(g, CHUNK)], b1.at[b], in_sems[b]),
            pltpu.async_copy(idx2_hbm.at[pl.ds(g, CHUNK)], b2.at[b], in_sems[b]),
            pltpu.async_copy(vals_hbm.at[pl.ds(g, CHUNK)], bv.at[b], in_sems[b]),
        ]

    pending_in = {0: issue_in(0, 0)}
    pending_sc = {}
    for k in range(NCHUNK):
        b = k % 2
        for cp in pending_in.pop(k):
            cp.wait()
        if k + 1 < NCHUNK:
            pending_in[k + 1] = issue_in(k + 1, (k + 1) % 2)
        if k - 2 in pending_sc:
            pending_sc.pop(k - 2).wait()

        lin_buf = lin_bufs[b]
        val_buf = val_bufs[b]

        @pl.loop(0, CHUNK, step=16)
        def _(co):
            sl = pl.ds(co, 16)
            m = b0[b, sl] == SLICE_IDX
            lin = b1[b, sl] * D2 + b2[b, sl]
            val = jnp.where(m, bv[b, sl], 0.0)
            lin_buf[sl] = lin
            val_buf[sl] = val

        pending_sc[k] = pltpu.async_copy(
            val_buf, accum.at[lin_buf], sc_sems[b], add=True)

    for h in pending_sc.values():
        h.wait()

    plsc.subcore_barrier()

    pltpu.sync_copy(accum.at[pl.ds(s * PER_SUB, PER_SUB)],
                    out_hbm.at[pl.ds(c * OUT_N + s * PER_SUB, PER_SUB)])


def _combine_body(a_ref, b_ref, o_ref):
    s = a_ref[...] + b_ref[...]
    o_ref[...] = s.reshape(o_ref.shape)


def kernel(idx0, idx1, idx2, values):
    partials = _sc_scatter(idx0, idx1, idx2, values)
    nrow = D1 * D2 // 128
    p = partials.reshape(2 * nrow, 128)
    nblk = 16
    rb = nrow // nblk
    out = pl.pallas_call(
        _combine_body,
        grid=(nblk,),
        in_specs=[
            pl.BlockSpec((rb, 128), lambda i: (i, 0)),
            pl.BlockSpec((rb, 128), lambda i: (i + nblk, 0)),
        ],
        out_specs=pl.BlockSpec((D1 // nblk, D2), lambda i: (i, 0)),
        out_shape=jax.ShapeDtypeStruct((D1, D2), jnp.float32),
    )(p, p)
    return out.reshape(1, D1, D2)


# speedup vs baseline: 43.4373x; 1.0790x over previous
import functools

import jax
import jax.numpy as jnp
from jax import lax
from jax.experimental import pallas as pl
from jax.experimental.pallas import tpu as pltpu
from jax.experimental.pallas import tpu_sc as plsc

NNZ = 1048576
D0, D1, D2 = 64, 4096, 256
SLICE_IDX = 10
OUT_N = D1 * D2

NUM_CORES = 2
NUM_SUBCORES = 16
NUM_TILES = NUM_CORES * NUM_SUBCORES
PER_TILE = NNZ // NUM_TILES
CHUNK = 4096
ROWS = CHUNK // 128
NCHUNK = PER_TILE // CHUNK
PER_SUB = OUT_N // NUM_SUBCORES
ZBUF = 4096

_mesh = plsc.VectorSubcoreMesh(core_axis_name="c", subcore_axis_name="s")


@functools.partial(
    pl.kernel,
    out_type=jax.ShapeDtypeStruct((2 * OUT_N,), jnp.float32),
    mesh=_mesh,
    scratch_types=[
        pltpu.VMEM((2, CHUNK), jnp.int32),
        pltpu.VMEM((2, CHUNK), jnp.int32),
        pltpu.VMEM((2, CHUNK), jnp.int32),
        pltpu.VMEM((2, CHUNK), jnp.float32),
        pltpu.VMEM((CHUNK,), jnp.int32),
        pltpu.VMEM((CHUNK,), jnp.int32),
        pltpu.VMEM((CHUNK,), jnp.float32),
        pltpu.VMEM((CHUNK,), jnp.float32),
        pltpu.VMEM((ZBUF,), jnp.float32),
        pltpu.VMEM_SHARED((OUT_N,), jnp.float32),
        pltpu.SemaphoreType.DMA,
        pltpu.SemaphoreType.DMA,
        pltpu.SemaphoreType.DMA,
        pltpu.SemaphoreType.DMA,
    ],
)
def _sc_scatter(idx0_hbm, idx1_hbm, idx2_hbm, vals_hbm, out_hbm,
                b0, b1, b2, bv, lin_a, lin_b, val_a, val_b, zbuf, accum,
                sem_a, sem_b, ssem_a, ssem_b):
    c = lax.axis_index("c")
    s = lax.axis_index("s")

    zero16 = jnp.zeros((16,), jnp.float32)

    @pl.loop(0, ZBUF, step=16)
    def _(i):
        zbuf[pl.ds(i, 16)] = zero16

    @pl.loop(0, PER_SUB, step=ZBUF)
    def _(k):
        pltpu.sync_copy(zbuf, accum.at[pl.ds(s * PER_SUB + k, ZBUF)])

    plsc.subcore_barrier()

    base = (c * NUM_SUBCORES + s) * PER_TILE
    in_sems = (sem_a, sem_b)
    sc_sems = (ssem_a, ssem_b)
    lin_bufs = (lin_a, lin_b)
    val_bufs = (val_a, val_b)

    def issue_in(k, b):
        g = base + k * CHUNK
        return [
            pltpu.async_copy(idx0_hbm.at[pl.ds(g, CHUNK)], b0.at[b], in_sems[b]),
            pltpu.async_copy(idx1_hbm.at[pl.ds(g, CHUNK)], b1.at[b], in_sems[b]),
            pltpu.async_copy(idx2_hbm.at[pl.ds(g, CHUNK)], b2.at[b], in_sems[b]),
            pltpu.async_copy(vals_hbm.at[pl.ds(g, CHUNK)], bv.at[b], in_sems[b]),
        ]

    pending_in = {0: issue_in(0, 0)}
    pending_sc = {}
    for k in range(NCHUNK):
        b = k % 2
        for cp in pending_in.pop(k):
            cp.wait()
        if k + 1 < NCHUNK:
            pending_in[k + 1] = issue_in(k + 1, (k + 1) % 2)
        if k - 2 in pending_sc:
            pending_sc.pop(k - 2).wait()

        lin_buf = lin_bufs[b]
        val_buf = val_bufs[b]

        @plsc.parallel_loop(0, CHUNK, step=16, unroll=8)
        def _(co):
            sl = pl.ds(co, 16)
            m = b0[b, sl] == SLICE_IDX
            lin = b1[b, sl] * D2 + b2[b, sl]
            val = jnp.where(m, bv[b, sl], 0.0)
            lin_buf[sl] = lin
            val_buf[sl] = val

        pending_sc[k] = pltpu.async_copy(
            val_buf, accum.at[lin_buf], sc_sems[b], add=True)

    for h in pending_sc.values():
        h.wait()

    plsc.subcore_barrier()

    pltpu.sync_copy(accum.at[pl.ds(s * PER_SUB, PER_SUB)],
                    out_hbm.at[pl.ds(c * OUT_N + s * PER_SUB, PER_SUB)])


def _combine_body(a_ref, b_ref, o_ref):
    s = a_ref[...] + b_ref[...]
    o_ref[...] = s.reshape(o_ref.shape)


def kernel(idx0, idx1, idx2, values):
    partials = _sc_scatter(idx0, idx1, idx2, values)
    nrow = D1 * D2 // 128
    p = partials.reshape(2 * nrow, 128)
    nblk = 16
    rb = nrow // nblk
    out = pl.pallas_call(
        _combine_body,
        grid=(nblk,),
        in_specs=[
            pl.BlockSpec((rb, 128), lambda i: (i, 0)),
            pl.BlockSpec((rb, 128), lambda i: (i + nblk, 0)),
        ],
        out_specs=pl.BlockSpec((D1 // nblk, D2), lambda i: (i, 0)),
        out_shape=jax.ShapeDtypeStruct((D1, D2), jnp.float32),
    )(p, p)
    return out.reshape(1, D1, D2)


# speedup vs baseline: 43.7357x; 1.0069x over previous
import dataclasses
import functools

import jax
import jax.numpy as jnp
from jax import lax
from jax.experimental import pallas as pl
from jax.experimental.pallas import tpu as pltpu
from jax.experimental.pallas import tpu_sc as plsc

NNZ = 1048576
D0, D1, D2 = 64, 4096, 256
SLICE_IDX = 10
OUT_N = D1 * D2

NUM_CORES = 2
NUM_SUBCORES = 16
NUM_TILES = NUM_CORES * NUM_SUBCORES
PER_TILE = NNZ // NUM_TILES
CHUNK = 4096
ROWS = CHUNK // 128
NCHUNK = PER_TILE // CHUNK
PER_SUB = OUT_N // NUM_SUBCORES
ZBUF = 4096

_mesh = plsc.VectorSubcoreMesh(core_axis_name="c", subcore_axis_name="s")

_cp = pltpu.CompilerParams()
if "needs_layout_passes" in pltpu.CompilerParams.__dataclass_fields__:
    _cp = dataclasses.replace(_cp, needs_layout_passes=False)


@functools.partial(
    pl.kernel,
    out_type=jax.ShapeDtypeStruct((2 * OUT_N,), jnp.float32),
    mesh=_mesh,
    compiler_params=_cp,
    scratch_types=[
        pltpu.VMEM((2, CHUNK), jnp.int32),
        pltpu.VMEM((2, CHUNK), jnp.int32),
        pltpu.VMEM((2, CHUNK), jnp.int32),
        pltpu.VMEM((2, CHUNK), jnp.float32),
        pltpu.VMEM((CHUNK + 128,), jnp.int32),
        pltpu.VMEM((CHUNK + 128,), jnp.float32),
        pltpu.VMEM((ZBUF,), jnp.float32),
        pltpu.VMEM_SHARED((OUT_N,), jnp.float32),
        pltpu.SemaphoreType.DMA,
        pltpu.SemaphoreType.DMA,
        pltpu.SemaphoreType.DMA,
        pltpu.SemaphoreType.DMA,
    ],
)
def _sc_scatter(idx0_hbm, idx1_hbm, idx2_hbm, vals_hbm, out_hbm,
                b0, b1, b2, bv, lin_buf, val_buf, zbuf, accum,
                sem_a, sem_b, ssem_a, ssem_b):
    c = lax.axis_index("c")
    s = lax.axis_index("s")

    zero16 = jnp.zeros((16,), jnp.float32)

    @pl.loop(0, ZBUF, step=16)
    def _(i):
        zbuf[pl.ds(i, 16)] = zero16

    @pl.loop(0, PER_SUB, step=ZBUF)
    def _(k):
        pltpu.sync_copy(zbuf, accum.at[pl.ds(s * PER_SUB + k, ZBUF)])

    plsc.subcore_barrier()

    base = (c * NUM_SUBCORES + s) * PER_TILE
    in_sems = (sem_a, sem_b)
    iota16 = lax.iota(jnp.int32, 16)
    zeros16_i = jnp.zeros((16,), jnp.int32)
    zeros16_f = jnp.zeros((16,), jnp.float32)

    def issue_in(k, b):
        g = base + k * CHUNK
        return [
            pltpu.async_copy(idx0_hbm.at[pl.ds(g, CHUNK)], b0.at[b], in_sems[b]),
            pltpu.async_copy(idx1_hbm.at[pl.ds(g, CHUNK)], b1.at[b], in_sems[b]),
            pltpu.async_copy(idx2_hbm.at[pl.ds(g, CHUNK)], b2.at[b], in_sems[b]),
            pltpu.async_copy(vals_hbm.at[pl.ds(g, CHUNK)], bv.at[b], in_sems[b]),
        ]

    pending_in = {0: issue_in(0, 0)}
    for k in range(NCHUNK):
        b = k % 2
        for cp in pending_in.pop(k):
            cp.wait()
        if k + 1 < NCHUNK:
            pending_in[k + 1] = issue_in(k + 1, (k + 1) % 2)
        @plsc.parallel_loop(0, CHUNK, step=16, unroll=8,
                            carry=jnp.zeros((16,), jnp.int32))
        def off_v(co, off):
            sl = pl.ds(co, 16)
            m = b0[b, sl] == SLICE_IDX
            cs = plsc.cumsum(m.astype(jnp.int32))
            cnt = plsc.all_reduce_population_count(m)
            lin = b1[b, sl] * D2 + b2[b, sl]
            val = bv[b, sl]
            dst = off + cs - 1
            plsc.store_scatter(lin_buf, [dst], lin, mask=m)
            plsc.store_scatter(val_buf, [dst], val, mask=m)
            return off + cnt

        for j in range(8):
            pad_idx = off_v + (iota16 + (j * 16))
            plsc.store_scatter(lin_buf, [pad_idx], zeros16_i)
            plsc.store_scatter(val_buf, [pad_idx], zeros16_f)

        cnt_sc = jnp.max(off_v)
        nblk = (cnt_sc + 127) >> 7

        @pl.loop(0, nblk)
        def _(j):
            o = j * 128
            pltpu.sync_copy(val_buf.at[pl.ds(o, 128)],
                            accum.at[lin_buf.at[pl.ds(o, 128)]], add=True)

    plsc.subcore_barrier()

    pltpu.sync_copy(accum.at[pl.ds(s * PER_SUB, PER_SUB)],
                    out_hbm.at[pl.ds(c * OUT_N + s * PER_SUB, PER_SUB)])


def _combine_body(a_ref, b_ref, o_ref):
    s = a_ref[...] + b_ref[...]
    o_ref[...] = s.reshape(o_ref.shape)


def kernel(idx0, idx1, idx2, values):
    partials = _sc_scatter(idx0, idx1, idx2, values)
    nrow = D1 * D2 // 128
    p = partials.reshape(2 * nrow, 128)
    nblk = 16
    rb = nrow // nblk
    out = pl.pallas_call(
        _combine_body,
        grid=(nblk,),
        in_specs=[
            pl.BlockSpec((rb, 128), lambda i: (i, 0)),
            pl.BlockSpec((rb, 128), lambda i: (i + nblk, 0)),
        ],
        out_specs=pl.BlockSpec((D1 // nblk, D2), lambda i: (i, 0)),
        out_shape=jax.ShapeDtypeStruct((D1, D2), jnp.float32),
    )(p, p)
    return out.reshape(1, D1, D2)
